# baseline (speedup 1.0000x reference)
import functools

import jax
import jax.numpy as jnp
from jax import lax
from jax.experimental import pallas as pl
from jax.experimental.pallas import tpu as pltpu
from jax.experimental.pallas import tpu_sc as plsc

D = 32
NC = 2
NS = 16
NW = NC * NS
SUB = 512
TB = 8
BL = 64
NF = 8
SPF = 25


@jax.jit
def _gather_native(x, weight):
    n_b, n_t = x.shape
    n_sub = (n_b * n_t) // SUB
    per_w = n_sub // NW
    mesh = plsc.VectorSubcoreMesh(core_axis_name="c", subcore_axis_name="s")

    @functools.partial(
        pl.kernel,
        mesh=mesh,
        out_type=jax.ShapeDtypeStruct((n_t, 4, 128, 8, 128), jnp.float32),
        scratch_types=[
            pltpu.VMEM((2, BL, n_t), jnp.int32),
            pltpu.VMEM((2, SUB), jnp.int32),
            pltpu.VMEM((2, SUB, D), jnp.float32),
            pltpu.VMEM((2, TB, 4, 8, BL), jnp.float32),
            [pltpu.SemaphoreType.DMA],
            [pltpu.SemaphoreType.DMA] * 2,
            [pltpu.SemaphoreType.DMA] * 2,
        ],
        compiler_params=pltpu.CompilerParams(
            use_tc_tiling_on_sc=False, needs_layout_passes=False
        ),
    )
    def k(x_hbm, table_hbm, out_hbm, idx_v, off_v, rows_v, dst_v,
          sem_i, sem_g, sem_o):
        wid = lax.axis_index("s") * NC + lax.axis_index("c")
        b0 = wid * (4 * 128)
        iota16 = lax.iota(jnp.int32, 16)

        def fetch_start(f):
            pltpu.async_copy(
                x_hbm.at[pl.ds(b0 + f * BL, BL), :],
                idx_v.at[f % 2],
                sem_i[0],
            )

        def fetch_wait():
            pltpu.make_async_copy(
                x_hbm.at[pl.ds(0, BL), :], idx_v.at[0], sem_i[0]
            ).wait()

        def idx_transpose(n, nb):
            src = idx_v.at[(n // SPF) % 2]
            tcol0 = (n % SPF) * TB
            for g in range(4):
                base = g * 16
                vs = []
                for t0 in range(TB):
                    tc = (t0 + iota16) & (TB - 1)
                    vs.append(
                        plsc.load_gather(src, [iota16 + base, tcol0 + tc])
                    )
                for t0 in range(TB):
                    tc = (t0 + iota16) & (TB - 1)
                    plsc.store_scatter(
                        off_v.at[nb], [tc * BL + base + iota16], vs[t0]
                    )

        def gather_start(b):
            pltpu.async_copy(
                table_hbm.at[off_v.at[b]], rows_v.at[b], sem_g[b]
            )

        def gather_wait(b):
            pltpu.make_async_copy(
                table_hbm.at[off_v.at[b]], rows_v.at[b], sem_g[b]
            ).wait()

        def transpose(b):
            rows = rows_v.at[b]
            for tdr in range(TB):
                base_row = tdr * BL

                def dbody(d0, carry):
                    cidx = (d0 + iota16) & (D - 1)
                    dtrv = cidx >> 3
                    ddrv = cidx & 7
                    vs = []
                    for g in range(4):
                        ridx = iota16 + (base_row + g * 16)
                        vs.append(plsc.load_gather(rows, [ridx, cidx]))
                    for g in range(4):
                        plsc.store_scatter(
                            dst_v.at[b, tdr],
                            [dtrv, ddrv, iota16 + g * 16],
                            vs[g],
                        )
                    return carry

                lax.fori_loop(0, D, dbody, 0)

        def out_start(n, b):
            btc = wid * 4 + n // 50
            bh = (n % 50) // SPF
            ttr = n % SPF
            pltpu.async_copy(
                dst_v.at[b],
                out_hbm.at[
                    pl.ds(ttr * TB, TB), :, btc, :, pl.ds(bh * BL, BL)
                ],
                sem_o[b],
            )

        def out_wait(b):
            pltpu.make_async_copy(
                dst_v.at[b],
                out_hbm.at[pl.ds(0, TB), :, 0, :, pl.ds(0, BL)],
                sem_o[b],
            ).wait()

        fetch_start(0)
        fetch_start(1)
        fetch_wait()
        idx_transpose(0, 0)
        gather_start(0)

        def body(n, nb):
            gather_wait(nb)

            @pl.when(n + 1 < per_w)
            def _():
                @pl.when((n + 1) % SPF == 0)
                def _():
                    m = (n + 1) // SPF

                    @pl.when(m + 1 < NF)
                    def _():
                        fetch_start(m + 1)

                    fetch_wait()

                idx_transpose(n + 1, 1 - nb)
                gather_start(1 - nb)

            @pl.when(n >= 2)
            def _():
                out_wait(nb)

            transpose(nb)
            out_start(n, nb)

        def pair(p, carry):
            body(2 * p, 0)
            body(2 * p + 1, 1)
            return carry

        lax.fori_loop(0, per_w // 2, pair, 0)

        out_wait(0)
        out_wait(1)

    return k(x, weight)


def kernel(x, weight):
    rows, cols = x.shape
    out5 = _gather_native(x.astype(jnp.int32), weight)
    return out5.transpose(2, 4, 0, 1, 3).reshape(rows, cols, D)
